# baseline (speedup 1.0000x reference)
import functools
import jax
import jax.numpy as jnp
from jax import lax
from jax.experimental import pallas as pl
from jax.experimental.pallas import tpu as pltpu
from jax.experimental.pallas import tpu_sc as plsc

D = 256
Q = 1024
K = 100000
TOPK = 16
INV_SQRT_D = 0.0625
NEG = -1.0e30

GSZ = 128
KBLK = 2048
NKB = 49
K_PAD = KBLK * NKB
NG = K_PAD // GSZ
NG_PAD = 896
GPB = KBLK // GSZ
QBLK = 128
QC = 128
NCQ = Q // QC
NQB = QC // QBLK

SC_NW = 32
QPW = QC // SC_NW

NEG_INIT = -3.0e38


def _scores_body(q_ref, k_ref, o_ref, g_ref):
    kb = pl.program_id(0)
    s = lax.dot_general(q_ref[...], k_ref[...],
                        (((1,), (1,)), ((), ())),
                        preferred_element_type=jnp.float32) * INV_SQRT_D
    col = kb * KBLK + lax.broadcasted_iota(jnp.int32, (QBLK, KBLK), 1)
    s = jnp.where(col < K, s, NEG)
    o_ref[...] = s
    g_ref[...] = jnp.max(s.reshape(QBLK, GPB, GSZ), axis=2).T


def _scores(queries, keys):
    return pl.pallas_call(
        _scores_body,
        grid=(NKB, NQB),
        in_specs=[
            pl.BlockSpec((QBLK, D), lambda kb, qb: (qb, 0)),
            pl.BlockSpec((KBLK, D), lambda kb, qb: (kb, 0)),
        ],
        out_specs=[
            pl.BlockSpec((QBLK, KBLK), lambda kb, qb: (qb, kb)),
            pl.BlockSpec((GPB, QBLK), lambda kb, qb: (kb, qb)),
        ],
        out_shape=[
            jax.ShapeDtypeStruct((QC, K_PAD), jnp.float32),
            jax.ShapeDtypeStruct((NG, QC), jnp.float32),
        ],
    )(queries, keys)


def _transpose_body(x_ref, o_ref):
    o_ref[...] = jnp.concatenate(
        [x_ref[...].T, jnp.full((QBLK, NG_PAD - NG), NEG, jnp.float32)], axis=1)


def _transpose_gmax(gmax_t):
    return pl.pallas_call(
        _transpose_body,
        grid=(NQB,),
        in_specs=[pl.BlockSpec((NG, QBLK), lambda qb: (0, qb))],
        out_specs=pl.BlockSpec((QBLK, NG_PAD), lambda qb: (qb, 0)),
        out_shape=jax.ShapeDtypeStruct((QC, NG_PAD), jnp.float32),
    )(gmax_t)


def _miras_body(q_ref, bf_ref, cf_ref, df_ref, bd_ref, cd_ref, dd_ref,
                ml_ref, o_ref):
    q = q_ref[...]
    i = lax.broadcasted_iota(jnp.int32, (D, D), 0)
    j = lax.broadcasted_iota(jnp.int32, (D, D), 1)

    def tier(b, c, d2):
        wt = 0.1 * jnp.tanh(lax.dot_general(
            c, b, (((1,), (1,)), ((), ())), preferred_element_type=jnp.float32))
        wt = wt + jnp.where(i == j, d2, 0.0)
        return lax.dot_general(q, wt, (((1,), (0,)), ((), ())),
                               preferred_element_type=jnp.float32)

    v_f = tier(bf_ref[...], cf_ref[...], df_ref[...])
    v_d = tier(bd_ref[...], cd_ref[...], dd_ref[...])
    w = jax.nn.sigmoid(ml_ref[0, 0])
    o_ref[...] = w * v_f + (1.0 - w) * v_d


def _miras(queries, B_f, C_f, D_f, B_d, C_d, D_d, mix_logit):
    return pl.pallas_call(
        _miras_body,
        out_shape=jax.ShapeDtypeStruct((Q, D), jnp.float32),
    )(queries, B_f, C_f, D_f.reshape(1, D), B_d, C_d, D_d.reshape(1, D),
      mix_logit.reshape(1, 1))


def _merge(R, Ri, v, vi):
    cs, ci = plsc.sort_key_val(v, vi)
    csr = lax.rev(cs, (0,))
    cir = lax.rev(ci, (0,))
    keep = R >= csr
    nR = jnp.where(keep, R, csr)
    nI = jnp.where(keep, Ri, cir)
    return plsc.sort_key_val(nR, nI)


def _sc_topk_body(gmax_hbm, scores_hbm, values_hbm, mem_hbm, out_hbm,
                  gbuf, det_v, gidx_v, idx_v, rows_v, mem_v, out_v,
                  sem, gsem, dsem):
    wid = lax.axis_index("s") * 2 + lax.axis_index("c")
    qbase = wid * QPW
    lanes = jnp.arange(16, dtype=jnp.int32)

    pltpu.sync_copy(mem_hbm.at[pl.ds(qbase * D, QPW * D)], mem_v)

    def gstart(qi):
        pltpu.async_copy(gmax_hbm.at[pl.ds((qbase + qi) * NG_PAD, NG_PAD)],
                         gbuf.at[qi % 2], sem)

    def gwait(qi):
        pltpu.make_async_copy(gmax_hbm.at[pl.ds((qbase + qi) * NG_PAD, NG_PAD)],
                              gbuf.at[qi % 2], sem).wait()

    gstart(0)

    def qloop(qi, _):
        @pl.when(qi + 1 < QPW)
        def _():
            gstart(qi + 1)

        gwait(qi)
        slot = qi % 2

        def gscan(t, carry):
            R, Ri, thr = carry
            v = gbuf[slot, pl.ds(t * 16, 16)]
            msk = v > thr

            def do(cr):
                R, Ri, _ = cr
                R2, Ri2 = _merge(R, Ri, v, t * 16 + lanes)
                return R2, Ri2, jnp.broadcast_to(jnp.min(R2), (16,))

            return lax.cond(jnp.any(msk), do, lambda cr: cr, carry)

        init = (jnp.full((16,), NEG_INIT), jnp.zeros((16,), jnp.int32),
                jnp.full((16,), NEG_INIT))
        Rg, Gi, gthr = lax.fori_loop(0, NG_PAD // 16, gscan, init)

        q = qbase + qi
        gidx_v[...] = q * NG + Gi
        pltpu.async_copy(scores_hbm.at[gidx_v], det_v, dsem).wait()
        gsc = [jnp.sum(jnp.where(lanes == jj, Gi, 0)) * GSZ
               for jj in range(TOPK)]

        carry = (jnp.full((16,), NEG_INIT), jnp.zeros((16,), jnp.int32),
                 jnp.full((16,), NEG_INIT))
        for jj in reversed(range(TOPK)):
            colbase = gsc[jj]

            def dstep(sub, c2, jj=jj, colbase=colbase):
                R, Ri, thr = c2
                v = det_v[jj, pl.ds(sub * 16, 16)]
                msk = v > thr

                def do(c3):
                    R, Ri, _ = c3
                    R2, Ri2 = _merge(R, Ri, v, colbase + sub * 16 + lanes)
                    return R2, Ri2, jnp.broadcast_to(jnp.min(R2), (16,))

                return lax.cond(jnp.any(msk), do, lambda c3: c3, c2)

            carry = lax.fori_loop(0, GSZ // 16, dstep, carry)
        R, Ri, _ = carry

        mx = jnp.max(R)
        e = jnp.exp(R - mx)
        w = e / jnp.sum(e)
        idx_v[...] = Ri
        pltpu.async_copy(values_hbm.at[idx_v], rows_v, gsem).wait()
        wjs = [jnp.sum(jnp.where(lanes == jj, w, 0.0)) for jj in range(TOPK)]
        for cg in range(D // 16):
            acc = mem_v[pl.ds(qi * D + cg * 16, 16)]
            for jj in range(TOPK):
                acc = acc + wjs[jj] * rows_v[jj, pl.ds(cg * 16, 16)]
            out_v[pl.ds(qi * D + cg * 16, 16)] = acc
        return 0

    lax.fori_loop(0, QPW, qloop, 0)
    pltpu.sync_copy(out_v, out_hbm.at[pl.ds(qbase * D, QPW * D)])


@functools.lru_cache(maxsize=1)
def _build_sc_topk():
    return functools.partial(
        pl.kernel,
        out_type=jax.ShapeDtypeStruct((QC * D,), jnp.float32),
        mesh=plsc.VectorSubcoreMesh(core_axis_name="c", subcore_axis_name="s"),
        scratch_types=[
            pltpu.VMEM((2, NG_PAD), jnp.float32),
            pltpu.VMEM((TOPK, GSZ), jnp.float32),
            pltpu.VMEM((TOPK,), jnp.int32),
            pltpu.VMEM((TOPK,), jnp.int32),
            pltpu.VMEM((TOPK, D), jnp.float32),
            pltpu.VMEM((QPW * D,), jnp.float32),
            pltpu.VMEM((QPW * D,), jnp.float32),
            pltpu.SemaphoreType.DMA,
            pltpu.SemaphoreType.DMA,
            pltpu.SemaphoreType.DMA,
        ],
        compiler_params=pltpu.CompilerParams(needs_layout_passes=False),
    )(_sc_topk_body)


def kernel(queries, keys, values, B_f, C_f, D_f, B_d, C_d, D_d, mix_logit):
    mem = _miras(queries, B_f, C_f, D_f, B_d, C_d, D_d, mix_logit)
    sc = _build_sc_topk()
    outs = []
    for c in range(NCQ):
        qs = lax.slice(queries, (c * QC, 0), ((c + 1) * QC, D))
        scores, gmax_t = _scores(qs, keys)
        gmax = _transpose_gmax(gmax_t)
        mem_c = lax.slice(mem, (c * QC, 0), ((c + 1) * QC, D))
        out_c = sc(gmax.reshape(-1), scores.reshape(QC * NG, GSZ),
                   values, mem_c.reshape(-1))
        outs.append(out_c.reshape(QC, D))
    return jnp.concatenate(outs, axis=0)


# speedup vs baseline: 1.3107x; 1.3107x over previous
import functools
import jax
import jax.numpy as jnp
from jax import lax
from jax.experimental import pallas as pl
from jax.experimental.pallas import tpu as pltpu
from jax.experimental.pallas import tpu_sc as plsc

D = 256
Q = 1024
K = 100000
TOPK = 16
INV_SQRT_D = 0.0625
NEG = -1.0e30

GSZ = 128
KBLK = 2048
NKB = 49
K_PAD = KBLK * NKB
NG = K_PAD // GSZ
NG_PAD = 896
GPB = KBLK // GSZ
QBLK = 256
QC = 256
NCQ = Q // QC
NQB = QC // QBLK

SC_NW = 32
QPW = QC // SC_NW

NEG_INIT = -3.0e38


def _scores_body(q_ref, k_ref, o_ref, g_ref):
    kb = pl.program_id(0)
    s = lax.dot_general(q_ref[...], k_ref[...],
                        (((1,), (1,)), ((), ())),
                        preferred_element_type=jnp.float32) * INV_SQRT_D
    col = kb * KBLK + lax.broadcasted_iota(jnp.int32, (QBLK, KBLK), 1)
    s = jnp.where(col < K, s, NEG)
    o_ref[...] = s
    g_ref[...] = jnp.max(s.reshape(QBLK, GPB, GSZ), axis=2).T


def _scores(queries, keys):
    return pl.pallas_call(
        _scores_body,
        grid=(NKB, NQB),
        in_specs=[
            pl.BlockSpec((QBLK, D), lambda kb, qb: (qb, 0)),
            pl.BlockSpec((KBLK, D), lambda kb, qb: (kb, 0)),
        ],
        out_specs=[
            pl.BlockSpec((QBLK, KBLK), lambda kb, qb: (qb, kb)),
            pl.BlockSpec((GPB, QBLK), lambda kb, qb: (kb, qb)),
        ],
        out_shape=[
            jax.ShapeDtypeStruct((QC, K_PAD), jnp.float32),
            jax.ShapeDtypeStruct((NG, QC), jnp.float32),
        ],
    )(queries, keys)


def _transpose_body(x_ref, o_ref):
    o_ref[...] = jnp.concatenate(
        [x_ref[...].T, jnp.full((QBLK, NG_PAD - NG), NEG, jnp.float32)], axis=1)


def _transpose_gmax(gmax_t):
    return pl.pallas_call(
        _transpose_body,
        grid=(NQB,),
        in_specs=[pl.BlockSpec((NG, QBLK), lambda qb: (0, qb))],
        out_specs=pl.BlockSpec((QBLK, NG_PAD), lambda qb: (qb, 0)),
        out_shape=jax.ShapeDtypeStruct((QC, NG_PAD), jnp.float32),
    )(gmax_t)


def _miras_body(q_ref, bf_ref, cf_ref, df_ref, bd_ref, cd_ref, dd_ref,
                ml_ref, o_ref):
    q = q_ref[...]
    i = lax.broadcasted_iota(jnp.int32, (D, D), 0)
    j = lax.broadcasted_iota(jnp.int32, (D, D), 1)

    def tier(b, c, d2):
        wt = 0.1 * jnp.tanh(lax.dot_general(
            c, b, (((1,), (1,)), ((), ())), preferred_element_type=jnp.float32))
        wt = wt + jnp.where(i == j, d2, 0.0)
        return lax.dot_general(q, wt, (((1,), (0,)), ((), ())),
                               preferred_element_type=jnp.float32)

    v_f = tier(bf_ref[...], cf_ref[...], df_ref[...])
    v_d = tier(bd_ref[...], cd_ref[...], dd_ref[...])
    w = jax.nn.sigmoid(ml_ref[0, 0])
    o_ref[...] = w * v_f + (1.0 - w) * v_d


def _miras(queries, B_f, C_f, D_f, B_d, C_d, D_d, mix_logit):
    return pl.pallas_call(
        _miras_body,
        out_shape=jax.ShapeDtypeStruct((Q, D), jnp.float32),
    )(queries, B_f, C_f, D_f.reshape(1, D), B_d, C_d, D_d.reshape(1, D),
      mix_logit.reshape(1, 1))


def _merge(R, Ri, v, vi):
    cs, ci = plsc.sort_key_val(v, vi)
    csr = lax.rev(cs, (0,))
    cir = lax.rev(ci, (0,))
    keep = R >= csr
    nR = jnp.where(keep, R, csr)
    nI = jnp.where(keep, Ri, cir)
    return plsc.sort_key_val(nR, nI)


def _sc_topk_body(gmax_hbm, scores_hbm, values_hbm, mem_hbm, out_hbm,
                  gbuf, det_v, gidx_v, idx_v, rows_v, mem_v, out_v,
                  sem, gsem, dsem):
    wid = lax.axis_index("s") * 2 + lax.axis_index("c")
    qbase = wid * QPW
    lanes = jnp.arange(16, dtype=jnp.int32)

    pltpu.sync_copy(mem_hbm.at[pl.ds(qbase * D, QPW * D)], mem_v)

    def gstart(qi):
        pltpu.async_copy(gmax_hbm.at[pl.ds((qbase + qi) * NG_PAD, NG_PAD)],
                         gbuf.at[qi % 2], sem)

    def gwait(qi):
        pltpu.make_async_copy(gmax_hbm.at[pl.ds((qbase + qi) * NG_PAD, NG_PAD)],
                              gbuf.at[qi % 2], sem).wait()

    gstart(0)

    def qloop(qi, _):
        @pl.when(qi + 1 < QPW)
        def _():
            gstart(qi + 1)

        gwait(qi)
        slot = qi % 2

        def gscan(t, carry):
            R, Ri, thr = carry
            v = gbuf[slot, pl.ds(t * 16, 16)]
            msk = v > thr

            def do(cr):
                R, Ri, _ = cr
                R2, Ri2 = _merge(R, Ri, v, t * 16 + lanes)
                return R2, Ri2, jnp.broadcast_to(jnp.min(R2), (16,))

            return lax.cond(jnp.any(msk), do, lambda cr: cr, carry)

        init = (jnp.full((16,), NEG_INIT), jnp.zeros((16,), jnp.int32),
                jnp.full((16,), NEG_INIT))
        Rg, Gi, gthr = lax.fori_loop(0, NG_PAD // 16, gscan, init)

        q = qbase + qi
        gidx_v[...] = q * NG + Gi
        pltpu.async_copy(scores_hbm.at[gidx_v], det_v, dsem).wait()
        gsc = [jnp.sum(jnp.where(lanes == jj, Gi, 0)) * GSZ
               for jj in range(TOPK)]

        carry = (jnp.full((16,), NEG_INIT), jnp.zeros((16,), jnp.int32),
                 jnp.full((16,), NEG_INIT))
        for jj in reversed(range(TOPK)):
            colbase = gsc[jj]

            def dstep(sub, c2, jj=jj, colbase=colbase):
                R, Ri, thr = c2
                v = det_v[jj, pl.ds(sub * 16, 16)]
                msk = v > thr

                def do(c3):
                    R, Ri, _ = c3
                    R2, Ri2 = _merge(R, Ri, v, colbase + sub * 16 + lanes)
                    return R2, Ri2, jnp.broadcast_to(jnp.min(R2), (16,))

                return lax.cond(jnp.any(msk), do, lambda c3: c3, c2)

            carry = lax.fori_loop(0, GSZ // 16, dstep, carry)
        R, Ri, _ = carry

        mx = jnp.max(R)
        e = jnp.exp(R - mx)
        w = e / jnp.sum(e)
        idx_v[...] = Ri
        pltpu.async_copy(values_hbm.at[idx_v], rows_v, gsem).wait()
        wjs = [jnp.sum(jnp.where(lanes == jj, w, 0.0)) for jj in range(TOPK)]
        for cg in range(D // 16):
            acc = mem_v[pl.ds(qi * D + cg * 16, 16)]
            for jj in range(TOPK):
                acc = acc + wjs[jj] * rows_v[jj, pl.ds(cg * 16, 16)]
            out_v[pl.ds(qi * D + cg * 16, 16)] = acc
        return 0

    lax.fori_loop(0, QPW, qloop, 0)
    pltpu.sync_copy(out_v, out_hbm.at[pl.ds(qbase * D, QPW * D)])


@functools.lru_cache(maxsize=1)
def _build_sc_topk():
    return functools.partial(
        pl.kernel,
        out_type=jax.ShapeDtypeStruct((QC * D,), jnp.float32),
        mesh=plsc.VectorSubcoreMesh(core_axis_name="c", subcore_axis_name="s"),
        scratch_types=[
            pltpu.VMEM((2, NG_PAD), jnp.float32),
            pltpu.VMEM((TOPK, GSZ), jnp.float32),
            pltpu.VMEM((TOPK,), jnp.int32),
            pltpu.VMEM((TOPK,), jnp.int32),
            pltpu.VMEM((TOPK, D), jnp.float32),
            pltpu.VMEM((QPW * D,), jnp.float32),
            pltpu.VMEM((QPW * D,), jnp.float32),
            pltpu.SemaphoreType.DMA,
            pltpu.SemaphoreType.DMA,
            pltpu.SemaphoreType.DMA,
        ],
        compiler_params=pltpu.CompilerParams(needs_layout_passes=False),
    )(_sc_topk_body)


def kernel(queries, keys, values, B_f, C_f, D_f, B_d, C_d, D_d, mix_logit):
    mem = _miras(queries, B_f, C_f, D_f, B_d, C_d, D_d, mix_logit)
    sc = _build_sc_topk()
    outs = []
    for c in range(NCQ):
        qs = lax.slice(queries, (c * QC, 0), ((c + 1) * QC, D))
        scores, gmax_t = _scores(qs, keys)
        gmax = _transpose_gmax(gmax_t)
        mem_c = lax.slice(mem, (c * QC, 0), ((c + 1) * QC, D))
        out_c = sc(gmax.reshape(-1), scores.reshape(QC * NG, GSZ),
                   values, mem_c.reshape(-1))
        outs.append(out_c.reshape(QC, D))
    return jnp.concatenate(outs, axis=0)


# speedup vs baseline: 1.3272x; 1.0126x over previous
import functools
import jax
import jax.numpy as jnp
from jax import lax
from jax.experimental import pallas as pl
from jax.experimental.pallas import tpu as pltpu
from jax.experimental.pallas import tpu_sc as plsc

D = 256
Q = 1024
K = 100000
TOPK = 16
INV_SQRT_D = 0.0625
NEG = -1.0e30

GSZ = 128
KBLK = 2048
NKB = 49
K_PAD = KBLK * NKB
NG = K_PAD // GSZ
NG_PAD = 896
GPB = KBLK // GSZ
QBLK = 256
QC = 256
NCQ = Q // QC
NQB = QC // QBLK

SC_NW = 32
QPW = QC // SC_NW

NEG_INIT = -3.0e38


def _scores_body(q_ref, k_ref, o_ref, g_ref):
    kb = pl.program_id(0)
    s = lax.dot_general(q_ref[...], k_ref[...],
                        (((1,), (1,)), ((), ())),
                        preferred_element_type=jnp.float32) * INV_SQRT_D
    col = kb * KBLK + lax.broadcasted_iota(jnp.int32, (QBLK, KBLK), 1)
    s = jnp.where(col < K, s, NEG)
    o_ref[...] = s
    g_ref[...] = jnp.max(s.reshape(QBLK, GPB, GSZ), axis=2).T


def _scores(queries, keys):
    return pl.pallas_call(
        _scores_body,
        grid=(NKB, NQB),
        in_specs=[
            pl.BlockSpec((QBLK, D), lambda kb, qb: (qb, 0)),
            pl.BlockSpec((KBLK, D), lambda kb, qb: (kb, 0)),
        ],
        out_specs=[
            pl.BlockSpec((QBLK, KBLK), lambda kb, qb: (qb, kb)),
            pl.BlockSpec((GPB, QBLK), lambda kb, qb: (kb, qb)),
        ],
        out_shape=[
            jax.ShapeDtypeStruct((QC, K_PAD), jnp.float32),
            jax.ShapeDtypeStruct((NG, QC), jnp.float32),
        ],
    )(queries, keys)


def _transpose_body(x_ref, o_ref):
    o_ref[...] = jnp.concatenate(
        [x_ref[...].T, jnp.full((QBLK, NG_PAD - NG), NEG, jnp.float32)], axis=1)


def _transpose_gmax(gmax_t):
    return pl.pallas_call(
        _transpose_body,
        grid=(NQB,),
        in_specs=[pl.BlockSpec((NG, QBLK), lambda qb: (0, qb))],
        out_specs=pl.BlockSpec((QBLK, NG_PAD), lambda qb: (qb, 0)),
        out_shape=jax.ShapeDtypeStruct((QC, NG_PAD), jnp.float32),
    )(gmax_t)


def _miras_body(q_ref, bf_ref, cf_ref, df_ref, bd_ref, cd_ref, dd_ref,
                ml_ref, o_ref):
    q = q_ref[...]
    i = lax.broadcasted_iota(jnp.int32, (D, D), 0)
    j = lax.broadcasted_iota(jnp.int32, (D, D), 1)

    def tier(b, c, d2):
        wt = 0.1 * jnp.tanh(lax.dot_general(
            c, b, (((1,), (1,)), ((), ())), preferred_element_type=jnp.float32))
        wt = wt + jnp.where(i == j, d2, 0.0)
        return lax.dot_general(q, wt, (((1,), (0,)), ((), ())),
                               preferred_element_type=jnp.float32)

    v_f = tier(bf_ref[...], cf_ref[...], df_ref[...])
    v_d = tier(bd_ref[...], cd_ref[...], dd_ref[...])
    w = jax.nn.sigmoid(ml_ref[0, 0])
    o_ref[...] = w * v_f + (1.0 - w) * v_d


def _miras(queries, B_f, C_f, D_f, B_d, C_d, D_d, mix_logit):
    return pl.pallas_call(
        _miras_body,
        out_shape=jax.ShapeDtypeStruct((Q, D), jnp.float32),
    )(queries, B_f, C_f, D_f.reshape(1, D), B_d, C_d, D_d.reshape(1, D),
      mix_logit.reshape(1, 1))


def _merge(R, Ri, v, vi):
    cs, ci = plsc.sort_key_val(v, vi)
    csr = lax.rev(cs, (0,))
    cir = lax.rev(ci, (0,))
    keep = R >= csr
    nR = jnp.where(keep, R, csr)
    nI = jnp.where(keep, Ri, cir)
    return plsc.sort_key_val(nR, nI)


def _sc_topk_body(gmax_hbm, scores_hbm, values_hbm, mem_hbm, out_hbm,
                  gbuf, det_v, gidx_v, idx_v, rows_v, mem_v, out_v,
                  sem, gsem, dsem):
    wid = lax.axis_index("s") * 2 + lax.axis_index("c")
    qbase = wid * QPW
    lanes = jnp.arange(16, dtype=jnp.int32)

    pltpu.sync_copy(mem_hbm.at[pl.ds(qbase * D, QPW * D)], mem_v)

    def gstart(qi):
        pltpu.async_copy(gmax_hbm.at[pl.ds((qbase + qi) * NG_PAD, NG_PAD)],
                         gbuf.at[qi % 2], sem)

    def gwait(qi):
        pltpu.make_async_copy(gmax_hbm.at[pl.ds((qbase + qi) * NG_PAD, NG_PAD)],
                              gbuf.at[qi % 2], sem).wait()

    gstart(0)

    def stage_a(qi):
        @pl.when(qi + 1 < QPW)
        def _():
            gstart(qi + 1)

        gwait(qi)
        slot = qi % 2

        def gscan(t, carry):
            R, Ri, thr = carry
            v = gbuf[slot, pl.ds(t * 16, 16)]
            msk = v > thr

            def do(cr):
                R, Ri, _ = cr
                R2, Ri2 = _merge(R, Ri, v, t * 16 + lanes)
                return R2, Ri2, jnp.broadcast_to(jnp.min(R2), (16,))

            return lax.cond(jnp.any(msk), do, lambda cr: cr, carry)

        init = (jnp.full((16,), NEG_INIT), jnp.zeros((16,), jnp.int32),
                jnp.full((16,), NEG_INIT))
        _, Gi, _ = lax.fori_loop(0, NG_PAD // 16, gscan, init)

        gidx_v[slot, :] = (qbase + qi) * NG + Gi
        pltpu.async_copy(scores_hbm.at[gidx_v.at[slot]], det_v.at[slot], dsem)
        return Gi

    def stage_b(qi, Gi):
        slot = qi % 2
        pltpu.make_async_copy(scores_hbm.at[gidx_v.at[slot]], det_v.at[slot],
                              dsem).wait()
        gsc = [jnp.sum(jnp.where(lanes == jj, Gi, 0)) * GSZ
               for jj in range(TOPK)]

        carry = (jnp.full((16,), NEG_INIT), jnp.zeros((16,), jnp.int32),
                 jnp.full((16,), NEG_INIT))
        for jj in reversed(range(TOPK)):
            colbase = gsc[jj]

            def dstep(sub, c2, jj=jj, colbase=colbase):
                R, Ri, thr = c2
                v = det_v[slot, jj, pl.ds(sub * 16, 16)]
                msk = v > thr

                def do(c3):
                    R, Ri, _ = c3
                    R2, Ri2 = _merge(R, Ri, v, colbase + sub * 16 + lanes)
                    return R2, Ri2, jnp.broadcast_to(jnp.min(R2), (16,))

                return lax.cond(jnp.any(msk), do, lambda c3: c3, c2)

            carry = lax.fori_loop(0, GSZ // 16, dstep, carry)
        R, Ri, _ = carry

        mx = jnp.max(R)
        e = jnp.exp(R - mx)
        w = e / jnp.sum(e)
        idx_v[slot, :] = Ri
        pltpu.async_copy(values_hbm.at[idx_v.at[slot]], rows_v.at[slot], gsem)
        return w

    def stage_c(qi, w):
        slot = qi % 2
        pltpu.make_async_copy(values_hbm.at[idx_v.at[slot]], rows_v.at[slot],
                              gsem).wait()
        wjs = [jnp.sum(jnp.where(lanes == jj, w, 0.0)) for jj in range(TOPK)]
        for cg in range(D // 16):
            acc = mem_v[pl.ds(qi * D + cg * 16, 16)]
            for jj in range(TOPK):
                acc = acc + wjs[jj] * rows_v[slot, jj, pl.ds(cg * 16, 16)]
            out_v[pl.ds(qi * D + cg * 16, 16)] = acc

    zi = jnp.zeros((16,), jnp.int32)
    zw = jnp.zeros((16,), jnp.float32)

    def body(i, carry):
        Gi_prev, w_prev = carry
        Gi_new = lax.cond(i < QPW, lambda: stage_a(i), lambda: zi)
        w_new = lax.cond((i >= 1) & (i <= QPW),
                         lambda: stage_b(i - 1, Gi_prev), lambda: zw)

        @pl.when(i >= 2)
        def _():
            stage_c(i - 2, w_prev)

        return Gi_new, w_new

    lax.fori_loop(0, QPW + 2, body, (zi, zw))
    pltpu.sync_copy(out_v, out_hbm.at[pl.ds(qbase * D, QPW * D)])


@functools.lru_cache(maxsize=1)
def _build_sc_topk():
    return functools.partial(
        pl.kernel,
        out_type=jax.ShapeDtypeStruct((QC * D,), jnp.float32),
        mesh=plsc.VectorSubcoreMesh(core_axis_name="c", subcore_axis_name="s"),
        scratch_types=[
            pltpu.VMEM((2, NG_PAD), jnp.float32),
            pltpu.VMEM((2, TOPK, GSZ), jnp.float32),
            pltpu.VMEM((2, TOPK), jnp.int32),
            pltpu.VMEM((2, TOPK), jnp.int32),
            pltpu.VMEM((2, TOPK, D), jnp.float32),
            pltpu.VMEM((QPW * D,), jnp.float32),
            pltpu.VMEM((QPW * D,), jnp.float32),
            pltpu.SemaphoreType.DMA,
            pltpu.SemaphoreType.DMA,
            pltpu.SemaphoreType.DMA,
        ],
        compiler_params=pltpu.CompilerParams(needs_layout_passes=False),
    )(_sc_topk_body)


def kernel(queries, keys, values, B_f, C_f, D_f, B_d, C_d, D_d, mix_logit):
    mem = _miras(queries, B_f, C_f, D_f, B_d, C_d, D_d, mix_logit)
    sc = _build_sc_topk()
    outs = []
    for c in range(NCQ):
        qs = lax.slice(queries, (c * QC, 0), ((c + 1) * QC, D))
        scores, gmax_t = _scores(qs, keys)
        gmax = _transpose_gmax(gmax_t)
        mem_c = lax.slice(mem, (c * QC, 0), ((c + 1) * QC, D))
        out_c = sc(gmax.reshape(-1), scores.reshape(QC * NG, GSZ),
                   values, mem_c.reshape(-1))
        outs.append(out_c.reshape(QC, D))
    return jnp.concatenate(outs, axis=0)
